# baseline (speedup 1.0000x reference)
import functools
import math

import jax
import jax.numpy as jnp
from jax import lax
from jax.experimental import pallas as pl
from jax.experimental.pallas import tpu as pltpu
from jax.experimental.pallas import tpu_sc as plsc

D = 64
B = 4096 * 200
NC = 2
NS = 16
NW = NC * NS
B_PER_W = B // NW
CHUNK = 512
NCHUNK = B_PER_W // CHUNK
SCALE = math.sqrt(D)
LPR = D // 16


def _body(idx_hbm, table_hbm, out_hbm, idx_v, rows_v, sem):
    wid = lax.axis_index("s") * NC + lax.axis_index("c")
    base = wid * B_PER_W
    pltpu.sync_copy(idx_hbm.at[pl.ds(base, B_PER_W)], idx_v)

    def chunk_body(g, _):
        pltpu.async_copy(
            table_hbm.at[idx_v.at[pl.ds(g * CHUNK, CHUNK)]], rows_v, sem
        ).wait()

        def row_body(r, _):
            for c in range(LPR):
                rows_v[r, pl.ds(c * 16, 16)] = (
                    rows_v[r, pl.ds(c * 16, 16)] * SCALE
                )
            return 0

        lax.fori_loop(0, CHUNK, row_body, 0)

        pltpu.sync_copy(rows_v, out_hbm.at[pl.ds(base + g * CHUNK, CHUNK)])
        return 0

    lax.fori_loop(0, NCHUNK, chunk_body, 0)


_gather_scale = functools.partial(
    pl.kernel,
    mesh=plsc.VectorSubcoreMesh(core_axis_name="c", subcore_axis_name="s"),
    out_type=jax.ShapeDtypeStruct((B, D), jnp.float32),
    scratch_types=[
        pltpu.VMEM((B_PER_W,), jnp.int32),
        pltpu.VMEM((CHUNK, D), jnp.float32),
        pltpu.SemaphoreType.DMA,
    ],
    compiler_params=pltpu.CompilerParams(use_tc_tiling_on_sc=False),
)(_body)


def kernel(node_ids, order_ids, value_ids, embedding_weight):
    idx = node_ids.reshape(-1)
    out = _gather_scale(idx, embedding_weight)
    return out.reshape(node_ids.shape + (D,))


# speedup vs baseline: 1.1185x; 1.1185x over previous
import functools
import math

import jax
import jax.numpy as jnp
from jax import lax
from jax.experimental import pallas as pl
from jax.experimental.pallas import tpu as pltpu
from jax.experimental.pallas import tpu_sc as plsc

D = 64
B = 4096 * 200
NC = 2
NS = 16
NW = NC * NS
B_PER_W = B // NW
CHUNK = 256
NCHUNK = B_PER_W // CHUNK
NBUF = 4
LOOKAHEAD = 2
NGROUP = NCHUNK // NBUF
SCALE = math.sqrt(D)
LPR = D // 16
ROW_UNROLL = 8


def _body(idx_hbm, table_hbm, out_hbm, idx_v,
          b0, b1, b2, b3, g0, g1, g2, g3, w0, w1, w2, w3):
    bufs = [b0, b1, b2, b3]
    gsems = [g0, g1, g2, g3]
    wsems = [w0, w1, w2, w3]
    wid = lax.axis_index("s") * NC + lax.axis_index("c")
    base = wid * B_PER_W
    pltpu.sync_copy(idx_hbm.at[pl.ds(base, B_PER_W)], idx_v)

    def start_gather(g, bi):
        pltpu.async_copy(
            table_hbm.at[idx_v.at[pl.ds(g * CHUNK, CHUNK)]], bufs[bi], gsems[bi]
        )

    def wait_gather(bi):
        pltpu.make_async_copy(
            table_hbm.at[idx_v.at[pl.ds(0, CHUNK)]], bufs[bi], gsems[bi]
        ).wait()

    def start_write(g, bi):
        pltpu.async_copy(
            bufs[bi], out_hbm.at[pl.ds(base + g * CHUNK, CHUNK)], wsems[bi]
        )

    def wait_write(bi):
        pltpu.make_async_copy(
            bufs[bi], out_hbm.at[pl.ds(base, CHUNK)], wsems[bi]
        ).wait()

    def scale_buf(bi):
        buf = bufs[bi]

        def row_block(r2, _):
            for u in range(ROW_UNROLL):
                r = r2 * ROW_UNROLL + u
                for c in range(LPR):
                    buf[r, pl.ds(c * 16, 16)] = buf[r, pl.ds(c * 16, 16)] * SCALE
            return 0

        lax.fori_loop(0, CHUNK // ROW_UNROLL, row_block, 0)

    for g in range(LOOKAHEAD):
        start_gather(g, g)

    def group_body(gg, _):
        for bi in range(NBUF):
            g = gg * NBUF + bi
            wait_gather(bi)
            scale_buf(bi)
            start_write(g, bi)
            gn = g + LOOKAHEAD
            bn = (bi + LOOKAHEAD) % NBUF

            @pl.when((gn >= NBUF) & (gn < NCHUNK))
            def _():
                wait_write(bn)

            @pl.when(gn < NCHUNK)
            def _():
                start_gather(gn, bn)

        return 0

    lax.fori_loop(0, NGROUP, group_body, 0)

    for bi in range(NBUF):
        wait_write(bi)


_gather_scale = functools.partial(
    pl.kernel,
    mesh=plsc.VectorSubcoreMesh(core_axis_name="c", subcore_axis_name="s"),
    out_type=jax.ShapeDtypeStruct((B, D), jnp.float32),
    scratch_types=(
        [pltpu.VMEM((B_PER_W,), jnp.int32)]
        + [pltpu.VMEM((CHUNK, D), jnp.float32) for _ in range(NBUF)]
        + [pltpu.SemaphoreType.DMA for _ in range(2 * NBUF)]
    ),
    compiler_params=pltpu.CompilerParams(use_tc_tiling_on_sc=False),
)(_body)


def kernel(node_ids, order_ids, value_ids, embedding_weight):
    idx = node_ids.reshape(-1)
    out = _gather_scale(idx, embedding_weight)
    return out.reshape(node_ids.shape + (D,))
